# baseline (speedup 1.0000x reference)
import jax
import jax.numpy as jnp
from jax import lax
from jax.experimental import pallas as pl
from jax.experimental.pallas import tpu as pltpu
from jax.experimental.pallas import tpu_sc as plsc

_BATCH = 16384
_HIST = 20
_DIM = 64
_NC = 2
_NS = 16
_NW = _NC * _NS
_ROWS_W = _BATCH // _NW
_Q = 128
_NQ = _ROWS_W // _Q
_NSTEP = _HIST * _NQ
_HALF = 256


def _gather_body(idx_hbm, table_hbm, out_hbm,
                 idx_vb, off_t, par_t, pstage, tstage, gsem, ssem):
    wid = lax.axis_index("s") * _NC + lax.axis_index("c")
    b0 = wid * _ROWS_W
    lanes = lax.iota(jnp.int32, 16)

    for ch in range(4):
        pltpu.sync_copy(idx_hbm.at[pl.ds(b0 + ch * _Q, _Q)], idx_vb)
        for h in range(_HIST):
            for l in range(8):
                rows = lanes + (16 * l)
                cols = jnp.full((16,), h, jnp.int32)
                v = plsc.load_gather(idx_vb, [rows, cols])
                off_t[h, pl.ds(ch * _Q + 16 * l, 16)] = v >> 1
                par_t[h, pl.ds(ch * _Q + 16 * l, 16)] = (v & 1) * _DIM

    def fire_gather(t):
        h = t >> 2
        q = lax.rem(t, 4)
        rung = lax.rem(t, 3)
        pltpu.async_copy(
            table_hbm.at[off_t.at[h, pl.ds(q * _Q, _Q)]],
            pstage.at[rung], gsem.at[rung])

    def wait_gather(t):
        h = t >> 2
        q = lax.rem(t, 4)
        rung = lax.rem(t, 3)
        pltpu.make_async_copy(
            table_hbm.at[off_t.at[h, pl.ds(q * _Q, _Q)]],
            pstage.at[rung], gsem.at[rung]).wait()

    def write_half(h, hf, buf):
        pltpu.async_copy(
            tstage.at[buf],
            out_hbm.at[h].at[:, pl.ds(b0 + hf * _HALF, _HALF)],
            ssem.at[buf])

    def wait_write(h, hf, buf):
        pltpu.make_async_copy(
            tstage.at[buf],
            out_hbm.at[h].at[:, pl.ds(b0 + hf * _HALF, _HALF)],
            ssem.at[buf]).wait()

    fire_gather(0)
    fire_gather(1)

    def step(t, carry):
        h = t >> 2
        q = lax.rem(t, 4)
        rung = lax.rem(t, 3)
        buf = lax.rem(t >> 1, 2)
        wait_gather(t)

        @pl.when(t + 2 < _NSTEP)
        def _():
            fire_gather(t + 2)

        @pl.when((lax.rem(t, 2) == 0) & (t >= 4))
        def _():
            wait_write(h - 1, q >> 1, buf)

        colb = lax.rem(q, 2) * _Q
        for g in range(8):
            b_rows = lanes + 16 * g
            par_vec = par_t[h, pl.ds(q * _Q + 16 * g, 16)]
            for c in range(_DIM):
                v = plsc.load_gather(pstage.at[rung], [b_rows, par_vec + c])
                tstage[buf, c, pl.ds(colb + 16 * g, 16)] = v

        @pl.when(lax.rem(t, 2) == 1)
        def _():
            write_half(h, q >> 1, buf)

        return carry

    lax.fori_loop(0, _NSTEP, step, 0)

    wait_write(_HIST - 1, 0, 0)
    wait_write(_HIST - 1, 1, 1)


@jax.jit
def kernel(indices, table):
    idx = indices.astype(jnp.int32)
    t2 = table.reshape(table.shape[0] // 2, 2 * table.shape[1])
    mesh = plsc.VectorSubcoreMesh(core_axis_name="c", subcore_axis_name="s")
    out = pl.kernel(
        _gather_body,
        out_type=jax.ShapeDtypeStruct((_HIST, _DIM, _BATCH), jnp.float32),
        mesh=mesh,
        scratch_types=[
            pltpu.VMEM((_Q, _HIST), jnp.int32),
            pltpu.VMEM((_HIST, _ROWS_W), jnp.int32),
            pltpu.VMEM((_HIST, _ROWS_W), jnp.int32),
            pltpu.VMEM((3, _Q, 2 * _DIM), jnp.float32),
            pltpu.VMEM((2, _DIM, _HALF), jnp.float32),
            pltpu.SemaphoreType.DMA((3,)),
            pltpu.SemaphoreType.DMA((2,)),
        ],
        compiler_params=pltpu.CompilerParams(
            use_tc_tiling_on_sc=True, needs_layout_passes=False),
    )(idx, t2)
    return out.transpose(2, 0, 1)


# speedup vs baseline: 1.1178x; 1.1178x over previous
import jax
import jax.numpy as jnp
from jax import lax
from jax.experimental import pallas as pl
from jax.experimental.pallas import tpu as pltpu
from jax.experimental.pallas import tpu_sc as plsc

_BATCH = 16384
_HIST = 20
_DIM = 64
_NC = 2
_NS = 16
_NW = _NC * _NS
_ROWS_W = _BATCH // _NW
_CH = 64
_NCH = _ROWS_W // _CH
_NP = _HIST // 2
_NSTEP = _NP * _NCH


def _gather_body(idx_hbm, table_hbm, out_hbm,
                 idx_vb, off_t, par_t, pstage, ystage, gsem, ssem):
    wid = lax.axis_index("s") * _NC + lax.axis_index("c")
    b0 = wid * _ROWS_W
    lanes = lax.iota(jnp.int32, 16)

    for ch in range(_NCH):
        pltpu.sync_copy(idx_hbm.at[pl.ds(b0 + ch * _CH, _CH)], idx_vb)
        for h in range(_HIST):
            for l in range(4):
                rows = lanes + (16 * l)
                cols = jnp.full((16,), h, jnp.int32)
                v = plsc.load_gather(idx_vb, [rows, cols])
                d = (h & 1) * _CH + 16 * l
                off_t[h >> 1, ch, pl.ds(d, 16)] = v >> 1
                par_t[h >> 1, ch, pl.ds(d, 16)] = (v & 1) * _DIM

    def fire_gather(t):
        pltpu.async_copy(
            table_hbm.at[off_t.at[t >> 3, lax.rem(t, _NCH)]],
            pstage.at[lax.rem(t, 2)], gsem.at[lax.rem(t, 2)])

    def wait_gather(t):
        pltpu.make_async_copy(
            table_hbm.at[off_t.at[t >> 3, lax.rem(t, _NCH)]],
            pstage.at[lax.rem(t, 2)], gsem.at[lax.rem(t, 2)]).wait()

    def write_tile(t):
        pltpu.async_copy(
            ystage.at[lax.rem(t, 2)],
            out_hbm.at[pl.ds(b0 + lax.rem(t, _NCH) * _CH, _CH),
                       pl.ds((t >> 3) * 2 * _DIM, 2 * _DIM)],
            ssem.at[lax.rem(t, 2)])

    def wait_tile(t):
        pltpu.make_async_copy(
            ystage.at[lax.rem(t, 2)],
            out_hbm.at[pl.ds(b0 + lax.rem(t, _NCH) * _CH, _CH),
                       pl.ds((t >> 3) * 2 * _DIM, 2 * _DIM)],
            ssem.at[lax.rem(t, 2)]).wait()

    fire_gather(0)

    def step(t, carry):
        p = t >> 3
        ch = lax.rem(t, _NCH)
        rung = lax.rem(t, 2)
        wait_gather(t)

        @pl.when(t + 1 < _NSTEP)
        def _():
            fire_gather(t + 1)

        @pl.when(t >= 2)
        def _():
            wait_tile(t - 2)

        for g in range(4):
            pv0 = par_t[p, ch, pl.ds(16 * g, 16)]
            pv1 = par_t[p, ch, pl.ds(_CH + 16 * g, 16)]
            for i in range(16):
                j = 16 * g + i
                p0 = pv0[i]
                p1 = pv1[i]
                for c in range(0, _DIM, 16):
                    ystage[rung, j, pl.ds(c, 16)] = \
                        pstage[rung, j, pl.ds(p0 + c, 16)]
                    ystage[rung, j, pl.ds(_DIM + c, 16)] = \
                        pstage[rung, _CH + j, pl.ds(p1 + c, 16)]

        write_tile(t)
        return carry

    lax.fori_loop(0, _NSTEP, step, 0)

    wait_tile(_NSTEP - 2)
    wait_tile(_NSTEP - 1)


@jax.jit
def kernel(indices, table):
    idx = indices.astype(jnp.int32)
    t2 = table.reshape(table.shape[0] // 2, 2 * table.shape[1])
    mesh = plsc.VectorSubcoreMesh(core_axis_name="c", subcore_axis_name="s")
    out = pl.kernel(
        _gather_body,
        out_type=jax.ShapeDtypeStruct((_BATCH, _HIST * _DIM), jnp.float32),
        mesh=mesh,
        scratch_types=[
            pltpu.VMEM((_CH, _HIST), jnp.int32),
            pltpu.VMEM((_NP, _NCH, 2 * _CH), jnp.int32),
            pltpu.VMEM((_NP, _NCH, 2 * _CH), jnp.int32),
            pltpu.VMEM((2, 2 * _CH, 2 * _DIM), jnp.float32),
            pltpu.VMEM((2, _CH, 2 * _DIM), jnp.float32),
            pltpu.SemaphoreType.DMA((2,)),
            pltpu.SemaphoreType.DMA((2,)),
        ],
        compiler_params=pltpu.CompilerParams(
            use_tc_tiling_on_sc=True, needs_layout_passes=False),
    )(idx, t2)
    return out.reshape(_BATCH, _HIST, _DIM)


# speedup vs baseline: 1.1223x; 1.0040x over previous
import jax
import jax.numpy as jnp
from jax import lax
from jax.experimental import pallas as pl
from jax.experimental.pallas import tpu as pltpu
from jax.experimental.pallas import tpu_sc as plsc

_BATCH = 16384
_HIST = 20
_DIM = 64
_NC = 2
_NS = 16
_NW = _NC * _NS
_ROWS_W = _BATCH // _NW
_CH = 64
_NCH = _ROWS_W // _CH
_NP = _HIST // 2
_NSTEP = _NP * _NCH


def _gather_body(idx_hbm, table_hbm, out_hbm,
                 idx_vb, off_t, par_t, pstage, ystage, gsem, ssem):
    wid = lax.axis_index("s") * _NC + lax.axis_index("c")
    b0 = wid * _ROWS_W
    lanes = lax.iota(jnp.int32, 16)

    for ch in range(_NCH):
        pltpu.sync_copy(idx_hbm.at[pl.ds(b0 + ch * _CH, _CH)], idx_vb)
        for h in range(_HIST):
            for l in range(4):
                rows = lanes + (16 * l)
                cols = jnp.full((16,), h, jnp.int32)
                v = plsc.load_gather(idx_vb, [rows, cols])
                d = (h & 1) * _CH + 16 * l
                off_t[h >> 1, ch, pl.ds(d, 16)] = v >> 1
                par_t[h >> 1, ch, pl.ds(d, 16)] = (v & 1) * _DIM

    def fire_gather(t):
        pltpu.async_copy(
            table_hbm.at[off_t.at[t >> 3, lax.rem(t, _NCH)]],
            pstage.at[lax.rem(t, 4)], gsem.at[lax.rem(t, 4)])

    def wait_gather(t):
        pltpu.make_async_copy(
            table_hbm.at[off_t.at[t >> 3, lax.rem(t, _NCH)]],
            pstage.at[lax.rem(t, 4)], gsem.at[lax.rem(t, 4)]).wait()

    def write_tile(t):
        pltpu.async_copy(
            ystage.at[lax.rem(t, 2)],
            out_hbm.at[pl.ds(b0 + lax.rem(t, _NCH) * _CH, _CH),
                       pl.ds((t >> 3) * 2 * _DIM, 2 * _DIM)],
            ssem.at[lax.rem(t, 2)])

    def wait_tile(t):
        pltpu.make_async_copy(
            ystage.at[lax.rem(t, 2)],
            out_hbm.at[pl.ds(b0 + lax.rem(t, _NCH) * _CH, _CH),
                       pl.ds((t >> 3) * 2 * _DIM, 2 * _DIM)],
            ssem.at[lax.rem(t, 2)]).wait()

    fire_gather(0)
    fire_gather(1)
    fire_gather(2)

    def step(t, carry):
        p = t >> 3
        ch = lax.rem(t, _NCH)
        rung = lax.rem(t, 4)
        ybuf = lax.rem(t, 2)
        wait_gather(t)

        @pl.when(t + 3 < _NSTEP)
        def _():
            fire_gather(t + 3)

        @pl.when(t >= 2)
        def _():
            wait_tile(t - 2)

        for g in range(4):
            pv0 = par_t[p, ch, pl.ds(16 * g, 16)]
            pv1 = par_t[p, ch, pl.ds(_CH + 16 * g, 16)]
            for i in range(16):
                j = 16 * g + i
                p0 = pv0[i]
                p1 = pv1[i]
                for c in range(0, _DIM, 16):
                    ystage[ybuf, j, pl.ds(c, 16)] = \
                        pstage[rung, j, pl.ds(p0 + c, 16)]
                    ystage[ybuf, j, pl.ds(_DIM + c, 16)] = \
                        pstage[rung, _CH + j, pl.ds(p1 + c, 16)]

        write_tile(t)
        return carry

    lax.fori_loop(0, _NSTEP, step, 0)

    wait_tile(_NSTEP - 2)
    wait_tile(_NSTEP - 1)


@jax.jit
def kernel(indices, table):
    idx = indices.astype(jnp.int32)
    t2 = table.reshape(table.shape[0] // 2, 2 * table.shape[1])
    mesh = plsc.VectorSubcoreMesh(core_axis_name="c", subcore_axis_name="s")
    out = pl.kernel(
        _gather_body,
        out_type=jax.ShapeDtypeStruct((_BATCH, _HIST * _DIM), jnp.float32),
        mesh=mesh,
        scratch_types=[
            pltpu.VMEM((_CH, _HIST), jnp.int32),
            pltpu.VMEM((_NP, _NCH, 2 * _CH), jnp.int32),
            pltpu.VMEM((_NP, _NCH, 2 * _CH), jnp.int32),
            pltpu.VMEM((4, 2 * _CH, 2 * _DIM), jnp.float32),
            pltpu.VMEM((2, _CH, 2 * _DIM), jnp.float32),
            pltpu.SemaphoreType.DMA((4,)),
            pltpu.SemaphoreType.DMA((2,)),
        ],
        compiler_params=pltpu.CompilerParams(
            use_tc_tiling_on_sc=True, needs_layout_passes=False),
    )(idx, t2)
    return out.reshape(_BATCH, _HIST, _DIM)
